# baseline (speedup 1.0000x reference)
import functools
import jax
import jax.numpy as jnp
from jax import lax
from jax.experimental import pallas as pl
from jax.experimental.pallas import tpu as pltpu
from jax.experimental.pallas import tpu_sc as plsc

NC = 2
NS = 16
NW = NC * NS

N = 10000
NP = 10016
E = 320000
D = 128
G = 16
EPS = 1e-5
F32 = jnp.float32
BF16 = jnp.bfloat16

EBLK = 512
EGRID = E // EBLK
NBLK = 2504
NGRID = NP // NBLK
M_ = 256


def _dot_bf16(a, b):
    return lax.dot_general(a.astype(BF16), b.astype(BF16),
                           (((1,), (0,)), ((), ())),
                           preferred_element_type=F32)


def _dot_f32(a, b):
    return lax.dot_general(a, b, (((1,), (0,)), ((), ())),
                           precision=lax.Precision.HIGHEST,
                           preferred_element_type=F32)


def _p1_body(xi_ref, xj_ref, w1_ref, b1_ref, a1_ref, st1_ref, acc_ref):
    i = pl.program_id(0)
    xi = xi_ref[...]
    d = xj_ref[...] - xi
    m = jnp.concatenate([xi, d], axis=1)
    a1 = jnp.maximum(_dot_bf16(m, w1_ref[...]) + b1_ref[0, :][None, :], 0.0)
    a1_ref[...] = a1

    ps = jnp.sum(a1.reshape(EBLK // 8, 8, D), axis=0)
    pq = jnp.sum((a1 * a1).reshape(EBLK // 8, 8, D), axis=0)

    @pl.when(i == 0)
    def _():
        acc_ref[...] = jnp.zeros_like(acc_ref)

    acc_ref[0:8, :] += ps
    acc_ref[8:16, :] += pq
    st1_ref[...] = jnp.concatenate([
        jnp.sum(acc_ref[0:8, :], axis=0, keepdims=True),
        jnp.sum(acc_ref[8:16, :], axis=0, keepdims=True),
        jnp.zeros((6, D), F32),
    ], axis=0)


def _tc_p1(xi, xj, w1, b1):
    return pl.pallas_call(
        _p1_body,
        grid=(EGRID,),
        in_specs=[
            pl.BlockSpec((EBLK, D), lambda i: (i, 0)),
            pl.BlockSpec((EBLK, D), lambda i: (i, 0)),
            pl.BlockSpec((2 * D, D), lambda i: (0, 0)),
            pl.BlockSpec((1, D), lambda i: (0, 0)),
        ],
        out_specs=[
            pl.BlockSpec((EBLK, D), lambda i: (i, 0)),
            pl.BlockSpec((8, D), lambda i: (0, 0)),
        ],
        out_shape=[
            jax.ShapeDtypeStruct((E, D), F32),
            jax.ShapeDtypeStruct((8, D), F32),
        ],
        scratch_shapes=[pltpu.VMEM((16, D), F32)],
    )(xi, xj, w1, b1.reshape(1, D))


def _p2_body(a1_ref, st1_ref, w2_ref, b2_ref, g1_ref, be1_ref,
             z2_ref, st2_ref, acc_ref):
    i = pl.program_id(0)

    s_ = st1_ref[0, :] / E
    q_ = st1_ref[1, :] / E
    den = jnp.sqrt(q_ - s_ * s_ + EPS)

    a1 = a1_ref[...]
    h1 = g1_ref[0, :][None, :] * (a1 - s_[None, :]) / den[None, :] \
        + be1_ref[0, :][None, :]
    z2 = _dot_bf16(h1, w2_ref[...]) + b2_ref[0, :][None, :]
    z2_ref[...] = z2

    a2 = jnp.maximum(z2, 0.0)
    ps = jnp.sum(a2.reshape(EBLK // 8, 8, D), axis=0)
    pq = jnp.sum((a2 * a2).reshape(EBLK // 8, 8, D), axis=0)

    @pl.when(i == 0)
    def _():
        acc_ref[...] = jnp.zeros_like(acc_ref)

    acc_ref[0:8, :] += ps
    acc_ref[8:16, :] += pq
    st2_ref[...] = jnp.concatenate([
        jnp.sum(acc_ref[0:8, :], axis=0, keepdims=True),
        jnp.sum(acc_ref[8:16, :], axis=0, keepdims=True),
        jnp.zeros((6, D), F32),
    ], axis=0)


def _tc_p2(a1, stats1, w2, b2, g1, be1):
    return pl.pallas_call(
        _p2_body,
        grid=(EGRID,),
        in_specs=[
            pl.BlockSpec((EBLK, D), lambda i: (i, 0)),
            pl.BlockSpec((8, D), lambda i: (0, 0)),
            pl.BlockSpec((D, D), lambda i: (0, 0)),
            pl.BlockSpec((1, D), lambda i: (0, 0)),
            pl.BlockSpec((1, D), lambda i: (0, 0)),
            pl.BlockSpec((1, D), lambda i: (0, 0)),
        ],
        out_specs=[
            pl.BlockSpec((EBLK, D), lambda i: (i, 0)),
            pl.BlockSpec((8, D), lambda i: (0, 0)),
        ],
        out_shape=[
            jax.ShapeDtypeStruct((E, D), F32),
            jax.ShapeDtypeStruct((8, D), F32),
        ],
        scratch_shapes=[pltpu.VMEM((16, D), F32)],
    )(a1, stats1, w2, b2.reshape(1, D), g1.reshape(1, D), be1.reshape(1, D))


def _fin_body(m_ref, st2_ref, g2_ref, be2_ref, x_ref):
    s_ = st2_ref[0, :] / E
    q_ = st2_ref[1, :] / E
    den = jnp.sqrt(q_ - s_ * s_ + EPS)
    m = m_ref[...]
    bn = g2_ref[0, :][None, :] * (jnp.maximum(m, 0.0) - s_[None, :]) \
        / den[None, :] + be2_ref[0, :][None, :]
    x_ref[...] = jnp.where(m > EMPTY_THRESH, bn, 0.0)


def _tc_fin(mz, stats2, g2, be2):
    return pl.pallas_call(
        _fin_body,
        grid=(NGRID,),
        in_specs=[
            pl.BlockSpec((NBLK, D), lambda i: (i, 0)),
            pl.BlockSpec((8, D), lambda i: (0, 0)),
            pl.BlockSpec((1, D), lambda i: (0, 0)),
            pl.BlockSpec((1, D), lambda i: (0, 0)),
        ],
        out_specs=pl.BlockSpec((NBLK, D), lambda i: (i, 0)),
        out_shape=jax.ShapeDtypeStruct((NP, D), F32),
    )(mz, stats2, g2.reshape(1, D), be2.reshape(1, D))


def _tail_main_body(m_ref, st2_ref, g2_ref, be2_ref, x1_ref, oh_ref,
                    lw_ref, lb_ref, seg_ref, st_ref, segacc_ref, acc_ref):
    i = pl.program_id(0)

    s_ = st2_ref[0, :] / E
    q_ = st2_ref[1, :] / E
    den = jnp.sqrt(q_ - s_ * s_ + EPS)
    m = m_ref[...]
    bn = g2_ref[0, :][None, :] * (jnp.maximum(m, 0.0) - s_[None, :]) \
        / den[None, :] + be2_ref[0, :][None, :]
    x2 = jnp.where(m > EMPTY_THRESH, bn, 0.0)

    cat = jnp.concatenate([x1_ref[...], x2], axis=1)
    opre = jnp.maximum(_dot_bf16(cat, lw_ref[...]) + lb_ref[0, :][None, :], 0.0)

    oh = oh_ref[...]
    rowmask = jnp.sum(oh, axis=1, keepdims=True)
    om = opre * rowmask

    ps = jnp.sum(om.reshape(NBLK // 8, 8, M_), axis=0)
    pq = jnp.sum((om * om).reshape(NBLK // 8, 8, M_), axis=0)
    seg = lax.dot_general(oh, om, (((0,), (0,)), ((), ())),
                          precision=lax.Precision.HIGHEST,
                          preferred_element_type=F32)
    cntp = jnp.sum(oh.reshape(NBLK // 8, 8, 128), axis=0)

    @pl.when(i == 0)
    def _():
        acc_ref[...] = jnp.zeros_like(acc_ref)
        segacc_ref[...] = jnp.zeros_like(segacc_ref)

    acc_ref[0:8, :] += ps
    acc_ref[8:16, :] += pq
    acc_ref[16:24, 0:128] += cntp
    segacc_ref[...] += seg[0:16, :]

    seg_ref[...] = segacc_ref[...]
    st_ref[...] = jnp.concatenate([
        jnp.sum(acc_ref[0:8, :], axis=0, keepdims=True),
        jnp.sum(acc_ref[8:16, :], axis=0, keepdims=True),
        jnp.sum(acc_ref[16:24, :], axis=0, keepdims=True),
        jnp.zeros((5, M_), F32),
    ], axis=0)


def _tc_tail_main(mz2, stats2, g2, be2, x1, oh, lw, lb):
    return pl.pallas_call(
        _tail_main_body,
        grid=(NGRID,),
        in_specs=[
            pl.BlockSpec((NBLK, D), lambda i: (i, 0)),
            pl.BlockSpec((8, D), lambda i: (0, 0)),
            pl.BlockSpec((1, D), lambda i: (0, 0)),
            pl.BlockSpec((1, D), lambda i: (0, 0)),
            pl.BlockSpec((NBLK, D), lambda i: (i, 0)),
            pl.BlockSpec((NBLK, 128), lambda i: (i, 0)),
            pl.BlockSpec((2 * D, M_), lambda i: (0, 0)),
            pl.BlockSpec((1, M_), lambda i: (0, 0)),
        ],
        out_specs=[
            pl.BlockSpec((16, M_), lambda i: (0, 0)),
            pl.BlockSpec((8, M_), lambda i: (0, 0)),
        ],
        out_shape=[
            jax.ShapeDtypeStruct((16, M_), F32),
            jax.ShapeDtypeStruct((8, M_), F32),
        ],
        scratch_shapes=[pltpu.VMEM((16, M_), F32), pltpu.VMEM((24, M_), F32)],
    )(mz2, stats2, g2.reshape(1, D), be2.reshape(1, D), x1, oh, lw,
      lb.reshape(1, M_))


def _tail_head_body(seg_ref, st_ref, lg_ref, lbe_ref,
                    m1w_ref, m1b_ref, m1g_ref, m1be_ref,
                    m2w_ref, m2b_ref, m2g_ref, m2be_ref,
                    mfw_ref, mfb_ref, y_ref):
    s_ = st_ref[0, :] / N
    q_ = st_ref[1, :] / N
    den = jnp.sqrt(q_ - s_ * s_ + EPS)
    cnt = st_ref[2, 0:16].reshape(16, 1)
    seg = seg_ref[...]
    mean_o = seg / jnp.maximum(cnt, 1.0)
    bnp = lg_ref[0, :][None, :] * (mean_o - s_[None, :]) / den[None, :] \
        + lbe_ref[0, :][None, :]
    pooled = jnp.where(cnt > 0.0, bnp, 0.0)

    def blk(xx, w, b, g, be):
        a = jnp.maximum(_dot_bf16(xx, w) + b[0, :][None, :], 0.0)
        mm = jnp.mean(a, axis=0, keepdims=True)
        vv = jnp.mean((a - mm) ** 2, axis=0, keepdims=True)
        return g[0, :][None, :] * (a - mm) / jnp.sqrt(vv + EPS) + be[0, :][None, :]

    h = blk(pooled, m1w_ref[...], m1b_ref, m1g_ref, m1be_ref)
    h = blk(h, m2w_ref[...], m2b_ref, m2g_ref, m2be_ref)
    hb = h.astype(BF16).astype(F32)
    wb = mfw_ref[...][:, 0].astype(BF16).astype(F32)
    y = jnp.sum(hb * wb[None, :], axis=1, keepdims=True) + mfb_ref[0, 0]
    y_ref[...] = jnp.broadcast_to(y, (16, 128))


def _tc_tail_head(seg, st, p):
    r1 = lambda a: a.reshape(1, -1)
    out = pl.pallas_call(
        _tail_head_body,
        out_shape=jax.ShapeDtypeStruct((16, 128), F32),
    )(seg, st, r1(p['l1_g']), r1(p['l1_be']),
      p['m1_W'], r1(p['m1_b']), r1(p['m1_g']), r1(p['m1_be']),
      p['m2_W'], r1(p['m2_b']), r1(p['m2_g']), r1(p['m2_be']),
      p['mf_W'], r1(p['mf_b']))
    return out[:, 0]


CB = 128
CHUNKS = E // CB
WPASS = (CHUNKS + NW - 1) // NW


def _gather_pair(table, src, dst):
    mesh = plsc.VectorSubcoreMesh(core_axis_name="c", subcore_axis_name="s")

    @functools.partial(
        pl.kernel,
        out_type=[jax.ShapeDtypeStruct((E, D), F32),
                  jax.ShapeDtypeStruct((E, D), F32)],
        mesh=mesh,
        scratch_types=[
            pltpu.VMEM((CB,), jnp.int32), pltpu.VMEM((CB,), jnp.int32),
            pltpu.VMEM((CB, D), F32), pltpu.VMEM((CB, D), F32),
            pltpu.SemaphoreType.DMA, pltpu.SemaphoreType.DMA,
        ],
    )
    def k(table_h, dst_h, src_h, xi_h, xj_h, idx_d, idx_s, bi, bj, sem1, sem2):
        wid = lax.axis_index("s") * NC + lax.axis_index("c")

        def body(t, carry):
            cid = wid + NW * t

            @pl.when(cid < CHUNKS)
            def _():
                base = cid * CB
                pltpu.sync_copy(dst_h.at[pl.ds(base, CB)], idx_d)
                pltpu.sync_copy(src_h.at[pl.ds(base, CB)], idx_s)
                c1 = pltpu.async_copy(table_h.at[idx_d], bi, sem1)
                c2 = pltpu.async_copy(table_h.at[idx_s], bj, sem2)
                c1.wait()
                c2.wait()
                pltpu.sync_copy(bi, xi_h.at[pl.ds(base, CB)])
                pltpu.sync_copy(bj, xj_h.at[pl.ds(base, CB)])

            return carry

        lax.fori_loop(0, WPASS, body, 0)

    return k(table, dst, src)


NPW = NP // NW
EPW = E // NW
CAPW = 10240
PCAPB = 1040
FLUSH = 1024
NEG = -3.0e38
EMPTY_THRESH = -1.0e38


def _route(dst):
    mesh = plsc.VectorSubcoreMesh(core_axis_name="c", subcore_axis_name="s")
    iota16 = lambda: lax.broadcasted_iota(jnp.int32, (16,), 0)
    I32 = jnp.int32

    @functools.partial(
        pl.kernel,
        out_type=[
            jax.ShapeDtypeStruct((NW * NW * CAPW,), I32),
            jax.ShapeDtypeStruct((NW * NW * CAPW,), I32),
            jax.ShapeDtypeStruct((NW * NW * 16,), I32),
        ],
        mesh=mesh,
        scratch_types=[
            pltpu.VMEM((2000,), I32),
            pltpu.VMEM((NW * PCAPB,), I32),
            pltpu.VMEM((NW * PCAPB,), I32),
            pltpu.VMEM((NW * 16,), I32),
            pltpu.VMEM((NW * 16,), I32),
            pltpu.VMEM((16,), I32),
        ],
    )
    def k(dst_h, eix_h, dl_h, cnt_h, dbuf, se, sd, scnt, sfl, cbuf):
        sid = lax.axis_index("s") * NC + lax.axis_index("c")
        zero16 = jnp.zeros((16,), I32)
        trash16 = jnp.full((16,), NPW, I32)

        def initb(b, carry):
            for j in range(PCAPB // 16):
                se[pl.ds(b * PCAPB + 16 * j, 16)] = zero16
                sd[pl.ds(b * PCAPB + 16 * j, 16)] = trash16
            scnt[pl.ds(b * 16, 16)] = zero16
            sfl[pl.ds(b * 16, 16)] = zero16
            return carry
        lax.fori_loop(0, NW, initb, 0)

        def chunk(c, carry):
            cbase = pl.multiple_of(sid * EPW + c * 2000, 8)
            pltpu.sync_copy(dst_h.at[pl.ds(cbase, 2000)], dbuf)

            def vec(kk, carry):
                dv = dbuf[pl.ds(16 * kk, 16)]
                fv = dv.astype(jnp.float32) + 0.5
                bv = (fv * (1.0 / NPW)).astype(I32)
                dlv = dv - bv * NPW
                gbase = cbase + 16 * kk

                for l in range(16):
                    b_l = bv[l]
                    d_l = dlv[l]
                    cnt = scnt[pl.ds(b_l * 16, 16)][0]
                    se[pl.ds(b_l * PCAPB + cnt, 16)] = jnp.full((16,), gbase + l, I32)
                    sd[pl.ds(b_l * PCAPB + cnt, 16)] = jnp.full((16,), d_l, I32)
                    cnt = cnt + 1
                    scnt[pl.ds(b_l * 16, 16)] = jnp.full((16,), cnt, I32)

                    @pl.when(cnt >= FLUSH)
                    def _():
                        fl = sfl[pl.ds(b_l * 16, 16)][0]
                        base = pl.multiple_of((b_l * NW + sid) * CAPW + fl, 8)
                        pltpu.sync_copy(se.at[pl.ds(b_l * PCAPB, FLUSH)],
                                        eix_h.at[pl.ds(base, FLUSH)])
                        pltpu.sync_copy(sd.at[pl.ds(b_l * PCAPB, FLUSH)],
                                        dl_h.at[pl.ds(base, FLUSH)])
                        sfl[pl.ds(b_l * 16, 16)] = jnp.full((16,), fl + FLUSH, I32)
                        scnt[pl.ds(b_l * 16, 16)] = jnp.zeros((16,), I32)
                return carry

            return lax.fori_loop(0, 2000 // 16, vec, carry)

        lax.fori_loop(0, EPW // 2000, chunk, 0)

        def finb(b, carry):
            cnt = scnt[pl.ds(b * 16, 16)][0]
            fl = sfl[pl.ds(b * 16, 16)][0]
            base = pl.multiple_of((b * NW + sid) * CAPW + fl, 8)
            pltpu.sync_copy(se.at[pl.ds(b * PCAPB, FLUSH)],
                            eix_h.at[pl.ds(base, FLUSH)])
            pltpu.sync_copy(sd.at[pl.ds(b * PCAPB, FLUSH)],
                            dl_h.at[pl.ds(base, FLUSH)])
            cbuf[pl.ds(0, 16)] = jnp.full((16,), fl + cnt, I32)
            pltpu.sync_copy(
                cbuf, cnt_h.at[pl.ds(pl.multiple_of((b * NW + sid) * 16, 8), 16)])
            return carry
        lax.fori_loop(0, NW, finb, 0)

    return k(dst)


def _scatter_max(z2, routed):
    eix, dl, cnts = routed
    mesh = plsc.VectorSubcoreMesh(core_axis_name="c", subcore_axis_name="s")
    I32 = jnp.int32

    @functools.partial(
        pl.kernel,
        out_type=jax.ShapeDtypeStruct((NP * D,), jnp.float32),
        mesh=mesh,
        scratch_types=[
            pltpu.VMEM(((NPW + 1) * D,), jnp.float32),
            pltpu.VMEM((512,), I32),
            pltpu.VMEM((256,), I32),
            pltpu.VMEM((256,), I32),
            pltpu.VMEM((256, D), jnp.float32),
            pltpu.SemaphoreType.DMA,
            pltpu.SemaphoreType.DMA,
        ],
    )
    def k(z2_h, eix_h, dl_h, cnt_h, out_h, nodebuf, cbuf, ebuf, dbuf, rows,
          sem, sem2):
        wid = lax.axis_index("s") * NC + lax.axis_index("c")
        ninf = jnp.full((16,), NEG, jnp.float32)

        def initrow(r, carry):
            nodebuf[pl.ds(16 * r, 16)] = ninf
            return carry
        lax.fori_loop(0, (NPW + 1) * D // 16, initrow, 0)

        pltpu.sync_copy(
            cnt_h.at[pl.ds(pl.multiple_of(wid * NW * 16, 8), 512)], cbuf)

        def seg(s_, carry):
            cnt = cbuf[pl.ds(s_ * 16, 16)][0]
            rbase = (wid * NW + s_) * CAPW
            nb = (cnt + 255) // 256
            lval = (cnt // FLUSH) * FLUSH + FLUSH

            def batch(f, carry):
                start = pl.multiple_of(
                    rbase + jnp.minimum(f * 256, lval - 256), 8)
                pltpu.sync_copy(eix_h.at[pl.ds(start, 256)], ebuf)
                pltpu.sync_copy(dl_h.at[pl.ds(start, 256)], dbuf)
                c1 = pltpu.async_copy(
                    z2_h.at[ebuf.at[pl.ds(0, 128)]], rows.at[pl.ds(0, 128)], sem)
                c2 = pltpu.async_copy(
                    z2_h.at[ebuf.at[pl.ds(128, 128)]], rows.at[pl.ds(128, 128)],
                    sem2)
                c1.wait()
                c2.wait()

                def grp(t, carry):
                    dlv = dbuf[pl.ds(16 * t, 16)]
                    for l in range(16):
                        off = dlv[l] * D
                        r_ = 16 * t + l
                        for j in range(8):
                            cur = nodebuf[pl.ds(off + 16 * j, 16)]
                            val = rows[r_, pl.ds(16 * j, 16)]
                            nodebuf[pl.ds(off + 16 * j, 16)] = jnp.maximum(cur, val)
                    return carry
                return lax.fori_loop(0, 16, grp, carry)

            return lax.fori_loop(0, nb, batch, carry)

        lax.fori_loop(0, NW, seg, 0)

        pltpu.sync_copy(
            nodebuf.at[pl.ds(0, NPW * D)],
            out_h.at[pl.ds(pl.multiple_of(wid * NPW * D, 8), NPW * D)])

    rows_shape = k(z2, eix, dl, cnts)
    return rows_shape.reshape(NP, D)


def _forward_impl(x, params, edge_index, batch):
    src = edge_index[0]
    dst = edge_index[1]
    xp = jnp.zeros((NP, D), F32).at[:N].set(x)
    oh = jnp.zeros((NP, 128), F32).at[jnp.arange(N), batch].set(1.0)

    routed = _route(dst)

    def conv(table, pre):
        xi, xj = _gather_pair(table, src, dst)
        a1, st1 = _tc_p1(xi, xj, params[pre + '_W1'], params[pre + '_b1'])
        z2, st2 = _tc_p2(a1, st1, params[pre + '_W2'], params[pre + '_b2'],
                         params[pre + '_g1'], params[pre + '_be1'])
        mz = _scatter_max(z2, routed)
        return mz, st2

    m1, st2a = conv(xp, 'c1')
    x1 = _tc_fin(m1, st2a, params['c1_g2'], params['c1_be2'])
    m2, st2b = conv(x1, 'c2')
    seg, st = _tc_tail_main(m2, st2b, params['c2_g2'], params['c2_be2'],
                            x1, oh, params['l1_W'], params['l1_b'])
    return _tc_tail_head(seg, st, params)


@jax.jit
def kernel(x, params, edge_index, batch):
    return _forward_impl(x, params, edge_index, batch)


# speedup vs baseline: 2.3636x; 2.3636x over previous
import functools
import jax
import jax.numpy as jnp
from jax import lax
from jax.experimental import pallas as pl
from jax.experimental.pallas import tpu as pltpu
from jax.experimental.pallas import tpu_sc as plsc

NC = 2
NS = 16
NW = NC * NS

N = 10000
NP = 10016
E = 320000
D = 128
G = 16
EPS = 1e-5
F32 = jnp.float32
BF16 = jnp.bfloat16

EBLK = 512
EGRID = E // EBLK
NBLK = 2504
NGRID = NP // NBLK
M_ = 256


def _dot_bf16(a, b):
    return lax.dot_general(a.astype(BF16), b.astype(BF16),
                           (((1,), (0,)), ((), ())),
                           preferred_element_type=F32)


def _dot_f32(a, b):
    return lax.dot_general(a, b, (((1,), (0,)), ((), ())),
                           precision=lax.Precision.HIGHEST,
                           preferred_element_type=F32)


def _p1_body(xi_ref, xj_ref, w1_ref, b1_ref, a1_ref, st1_ref, acc_ref):
    i = pl.program_id(0)
    xi = xi_ref[...]
    d = xj_ref[...] - xi
    m = jnp.concatenate([xi, d], axis=1)
    a1 = jnp.maximum(_dot_bf16(m, w1_ref[...]) + b1_ref[0, :][None, :], 0.0)
    a1_ref[...] = a1

    ps = jnp.sum(a1.reshape(EBLK // 8, 8, D), axis=0)
    pq = jnp.sum((a1 * a1).reshape(EBLK // 8, 8, D), axis=0)

    @pl.when(i == 0)
    def _():
        acc_ref[...] = jnp.zeros_like(acc_ref)

    acc_ref[0:8, :] += ps
    acc_ref[8:16, :] += pq
    st1_ref[...] = jnp.concatenate([
        jnp.sum(acc_ref[0:8, :], axis=0, keepdims=True),
        jnp.sum(acc_ref[8:16, :], axis=0, keepdims=True),
        jnp.zeros((6, D), F32),
    ], axis=0)


def _tc_p1(xi, xj, w1, b1):
    return pl.pallas_call(
        _p1_body,
        grid=(EGRID,),
        in_specs=[
            pl.BlockSpec((EBLK, D), lambda i: (i, 0)),
            pl.BlockSpec((EBLK, D), lambda i: (i, 0)),
            pl.BlockSpec((2 * D, D), lambda i: (0, 0)),
            pl.BlockSpec((1, D), lambda i: (0, 0)),
        ],
        out_specs=[
            pl.BlockSpec((EBLK, D), lambda i: (i, 0)),
            pl.BlockSpec((8, D), lambda i: (0, 0)),
        ],
        out_shape=[
            jax.ShapeDtypeStruct((E, D), F32),
            jax.ShapeDtypeStruct((8, D), F32),
        ],
        scratch_shapes=[pltpu.VMEM((16, D), F32)],
    )(xi, xj, w1, b1.reshape(1, D))


def _p2_body(a1_ref, st1_ref, w2_ref, b2_ref, g1_ref, be1_ref,
             z2_ref, st2_ref, acc_ref):
    i = pl.program_id(0)

    s_ = st1_ref[0, :] / E
    q_ = st1_ref[1, :] / E
    den = jnp.sqrt(q_ - s_ * s_ + EPS)

    a1 = a1_ref[...]
    h1 = g1_ref[0, :][None, :] * (a1 - s_[None, :]) / den[None, :] \
        + be1_ref[0, :][None, :]
    z2 = _dot_bf16(h1, w2_ref[...]) + b2_ref[0, :][None, :]
    z2_ref[...] = z2

    a2 = jnp.maximum(z2, 0.0)
    ps = jnp.sum(a2.reshape(EBLK // 8, 8, D), axis=0)
    pq = jnp.sum((a2 * a2).reshape(EBLK // 8, 8, D), axis=0)

    @pl.when(i == 0)
    def _():
        acc_ref[...] = jnp.zeros_like(acc_ref)

    acc_ref[0:8, :] += ps
    acc_ref[8:16, :] += pq
    st2_ref[...] = jnp.concatenate([
        jnp.sum(acc_ref[0:8, :], axis=0, keepdims=True),
        jnp.sum(acc_ref[8:16, :], axis=0, keepdims=True),
        jnp.zeros((6, D), F32),
    ], axis=0)


def _tc_p2(a1, stats1, w2, b2, g1, be1):
    return pl.pallas_call(
        _p2_body,
        grid=(EGRID,),
        in_specs=[
            pl.BlockSpec((EBLK, D), lambda i: (i, 0)),
            pl.BlockSpec((8, D), lambda i: (0, 0)),
            pl.BlockSpec((D, D), lambda i: (0, 0)),
            pl.BlockSpec((1, D), lambda i: (0, 0)),
            pl.BlockSpec((1, D), lambda i: (0, 0)),
            pl.BlockSpec((1, D), lambda i: (0, 0)),
        ],
        out_specs=[
            pl.BlockSpec((EBLK, D), lambda i: (i, 0)),
            pl.BlockSpec((8, D), lambda i: (0, 0)),
        ],
        out_shape=[
            jax.ShapeDtypeStruct((E, D), F32),
            jax.ShapeDtypeStruct((8, D), F32),
        ],
        scratch_shapes=[pltpu.VMEM((16, D), F32)],
    )(a1, stats1, w2, b2.reshape(1, D), g1.reshape(1, D), be1.reshape(1, D))


def _fin_body(m_ref, st2_ref, g2_ref, be2_ref, x_ref):
    s_ = st2_ref[0, :] / E
    q_ = st2_ref[1, :] / E
    den = jnp.sqrt(q_ - s_ * s_ + EPS)
    m = m_ref[...]
    bn = g2_ref[0, :][None, :] * (jnp.maximum(m, 0.0) - s_[None, :]) \
        / den[None, :] + be2_ref[0, :][None, :]
    x_ref[...] = jnp.where(m > EMPTY_THRESH, bn, 0.0)


def _tc_fin(mz, stats2, g2, be2):
    return pl.pallas_call(
        _fin_body,
        grid=(NGRID,),
        in_specs=[
            pl.BlockSpec((NBLK, D), lambda i: (i, 0)),
            pl.BlockSpec((8, D), lambda i: (0, 0)),
            pl.BlockSpec((1, D), lambda i: (0, 0)),
            pl.BlockSpec((1, D), lambda i: (0, 0)),
        ],
        out_specs=pl.BlockSpec((NBLK, D), lambda i: (i, 0)),
        out_shape=jax.ShapeDtypeStruct((NP, D), F32),
    )(mz, stats2, g2.reshape(1, D), be2.reshape(1, D))


def _tail_main_body(m_ref, st2_ref, g2_ref, be2_ref, x1_ref, oh_ref,
                    lw_ref, lb_ref, seg_ref, st_ref, segacc_ref, acc_ref):
    i = pl.program_id(0)

    s_ = st2_ref[0, :] / E
    q_ = st2_ref[1, :] / E
    den = jnp.sqrt(q_ - s_ * s_ + EPS)
    m = m_ref[...]
    bn = g2_ref[0, :][None, :] * (jnp.maximum(m, 0.0) - s_[None, :]) \
        / den[None, :] + be2_ref[0, :][None, :]
    x2 = jnp.where(m > EMPTY_THRESH, bn, 0.0)

    cat = jnp.concatenate([x1_ref[...], x2], axis=1)
    opre = jnp.maximum(_dot_bf16(cat, lw_ref[...]) + lb_ref[0, :][None, :], 0.0)

    oh = oh_ref[...]
    rowmask = jnp.sum(oh, axis=1, keepdims=True)
    om = opre * rowmask

    ps = jnp.sum(om.reshape(NBLK // 8, 8, M_), axis=0)
    pq = jnp.sum((om * om).reshape(NBLK // 8, 8, M_), axis=0)
    seg = lax.dot_general(oh, om, (((0,), (0,)), ((), ())),
                          precision=lax.Precision.HIGHEST,
                          preferred_element_type=F32)
    cntp = jnp.sum(oh.reshape(NBLK // 8, 8, 128), axis=0)

    @pl.when(i == 0)
    def _():
        acc_ref[...] = jnp.zeros_like(acc_ref)
        segacc_ref[...] = jnp.zeros_like(segacc_ref)

    acc_ref[0:8, :] += ps
    acc_ref[8:16, :] += pq
    acc_ref[16:24, 0:128] += cntp
    segacc_ref[...] += seg[0:16, :]

    seg_ref[...] = segacc_ref[...]
    st_ref[...] = jnp.concatenate([
        jnp.sum(acc_ref[0:8, :], axis=0, keepdims=True),
        jnp.sum(acc_ref[8:16, :], axis=0, keepdims=True),
        jnp.sum(acc_ref[16:24, :], axis=0, keepdims=True),
        jnp.zeros((5, M_), F32),
    ], axis=0)


def _tc_tail_main(mz2, stats2, g2, be2, x1, oh, lw, lb):
    return pl.pallas_call(
        _tail_main_body,
        grid=(NGRID,),
        in_specs=[
            pl.BlockSpec((NBLK, D), lambda i: (i, 0)),
            pl.BlockSpec((8, D), lambda i: (0, 0)),
            pl.BlockSpec((1, D), lambda i: (0, 0)),
            pl.BlockSpec((1, D), lambda i: (0, 0)),
            pl.BlockSpec((NBLK, D), lambda i: (i, 0)),
            pl.BlockSpec((NBLK, 128), lambda i: (i, 0)),
            pl.BlockSpec((2 * D, M_), lambda i: (0, 0)),
            pl.BlockSpec((1, M_), lambda i: (0, 0)),
        ],
        out_specs=[
            pl.BlockSpec((16, M_), lambda i: (0, 0)),
            pl.BlockSpec((8, M_), lambda i: (0, 0)),
        ],
        out_shape=[
            jax.ShapeDtypeStruct((16, M_), F32),
            jax.ShapeDtypeStruct((8, M_), F32),
        ],
        scratch_shapes=[pltpu.VMEM((16, M_), F32), pltpu.VMEM((24, M_), F32)],
    )(mz2, stats2, g2.reshape(1, D), be2.reshape(1, D), x1, oh, lw,
      lb.reshape(1, M_))


def _tail_head_body(seg_ref, st_ref, lg_ref, lbe_ref,
                    m1w_ref, m1b_ref, m1g_ref, m1be_ref,
                    m2w_ref, m2b_ref, m2g_ref, m2be_ref,
                    mfw_ref, mfb_ref, y_ref):
    s_ = st_ref[0, :] / N
    q_ = st_ref[1, :] / N
    den = jnp.sqrt(q_ - s_ * s_ + EPS)
    cnt = st_ref[2, 0:16].reshape(16, 1)
    seg = seg_ref[...]
    mean_o = seg / jnp.maximum(cnt, 1.0)
    bnp = lg_ref[0, :][None, :] * (mean_o - s_[None, :]) / den[None, :] \
        + lbe_ref[0, :][None, :]
    pooled = jnp.where(cnt > 0.0, bnp, 0.0)

    def blk(xx, w, b, g, be):
        a = jnp.maximum(_dot_bf16(xx, w) + b[0, :][None, :], 0.0)
        mm = jnp.mean(a, axis=0, keepdims=True)
        vv = jnp.mean((a - mm) ** 2, axis=0, keepdims=True)
        return g[0, :][None, :] * (a - mm) / jnp.sqrt(vv + EPS) + be[0, :][None, :]

    h = blk(pooled, m1w_ref[...], m1b_ref, m1g_ref, m1be_ref)
    h = blk(h, m2w_ref[...], m2b_ref, m2g_ref, m2be_ref)
    hb = h.astype(BF16).astype(F32)
    wb = mfw_ref[...][:, 0].astype(BF16).astype(F32)
    y = jnp.sum(hb * wb[None, :], axis=1, keepdims=True) + mfb_ref[0, 0]
    y_ref[...] = jnp.broadcast_to(y, (16, 128))


def _tc_tail_head(seg, st, p):
    r1 = lambda a: a.reshape(1, -1)
    out = pl.pallas_call(
        _tail_head_body,
        out_shape=jax.ShapeDtypeStruct((16, 128), F32),
    )(seg, st, r1(p['l1_g']), r1(p['l1_be']),
      p['m1_W'], r1(p['m1_b']), r1(p['m1_g']), r1(p['m1_be']),
      p['m2_W'], r1(p['m2_b']), r1(p['m2_g']), r1(p['m2_be']),
      p['mf_W'], r1(p['mf_b']))
    return out[:, 0]


CB = 128
CHUNKS = E // CB
WPASS = (CHUNKS + NW - 1) // NW


def _gather_pair(table, src, dst):
    mesh = plsc.VectorSubcoreMesh(core_axis_name="c", subcore_axis_name="s")

    @functools.partial(
        pl.kernel,
        out_type=[jax.ShapeDtypeStruct((E, D), F32),
                  jax.ShapeDtypeStruct((E, D), F32)],
        mesh=mesh,
        scratch_types=[
            pltpu.VMEM((CB,), jnp.int32), pltpu.VMEM((CB,), jnp.int32),
            pltpu.VMEM((CB, D), F32), pltpu.VMEM((CB, D), F32),
            pltpu.SemaphoreType.DMA, pltpu.SemaphoreType.DMA,
        ],
    )
    def k(table_h, dst_h, src_h, xi_h, xj_h, idx_d, idx_s, bi, bj, sem1, sem2):
        wid = lax.axis_index("s") * NC + lax.axis_index("c")

        def body(t, carry):
            cid = wid + NW * t

            @pl.when(cid < CHUNKS)
            def _():
                base = cid * CB
                pltpu.sync_copy(dst_h.at[pl.ds(base, CB)], idx_d)
                pltpu.sync_copy(src_h.at[pl.ds(base, CB)], idx_s)
                c1 = pltpu.async_copy(table_h.at[idx_d], bi, sem1)
                c2 = pltpu.async_copy(table_h.at[idx_s], bj, sem2)
                c1.wait()
                c2.wait()
                pltpu.sync_copy(bi, xi_h.at[pl.ds(base, CB)])
                pltpu.sync_copy(bj, xj_h.at[pl.ds(base, CB)])

            return carry

        lax.fori_loop(0, WPASS, body, 0)

    return k(table, dst, src)


NPW = NP // NW
EPW = E // NW
CAPW = 10240
PCAPB = 1040
FLUSH = 1024
NEG = -3.0e38
EMPTY_THRESH = -1.0e38


def _route(dst):
    mesh = plsc.VectorSubcoreMesh(core_axis_name="c", subcore_axis_name="s")
    iota16 = lambda: lax.broadcasted_iota(jnp.int32, (16,), 0)
    I32 = jnp.int32

    @functools.partial(
        pl.kernel,
        out_type=[
            jax.ShapeDtypeStruct((NW * NW * CAPW,), I32),
            jax.ShapeDtypeStruct((NW * NW * CAPW,), I32),
            jax.ShapeDtypeStruct((NW * NW * 16,), I32),
        ],
        mesh=mesh,
        scratch_types=[
            pltpu.VMEM((2000,), I32),
            pltpu.VMEM((NW * PCAPB,), I32),
            pltpu.VMEM((NW * PCAPB,), I32),
            pltpu.VMEM((NW * 16,), I32),
            pltpu.VMEM((NW * 16,), I32),
            pltpu.VMEM((16,), I32),
        ],
    )
    def k(dst_h, eix_h, dl_h, cnt_h, dbuf, se, sd, scnt, sfl, cbuf):
        sid = lax.axis_index("s") * NC + lax.axis_index("c")
        zero16 = jnp.zeros((16,), I32)
        trash16 = jnp.full((16,), NPW, I32)

        def initb(b, carry):
            for j in range(PCAPB // 16):
                se[pl.ds(b * PCAPB + 16 * j, 16)] = zero16
                sd[pl.ds(b * PCAPB + 16 * j, 16)] = trash16
            scnt[pl.ds(b * 16, 16)] = zero16
            sfl[pl.ds(b * 16, 16)] = zero16
            return carry
        lax.fori_loop(0, NW, initb, 0)

        def chunk(c, carry):
            cbase = pl.multiple_of(sid * EPW + c * 2000, 8)
            pltpu.sync_copy(dst_h.at[pl.ds(cbase, 2000)], dbuf)

            def vec(kk, carry):
                dv = dbuf[pl.ds(16 * kk, 16)]
                fv = dv.astype(jnp.float32) + 0.5
                bv = (fv * (1.0 / NPW)).astype(I32)
                dlv = dv - bv * NPW
                gbase = cbase + 16 * kk

                for l in range(16):
                    b_l = bv[l]
                    d_l = dlv[l]
                    cnt = scnt[pl.ds(b_l * 16, 16)][0]
                    se[pl.ds(b_l * PCAPB + cnt, 16)] = jnp.full((16,), gbase + l, I32)
                    sd[pl.ds(b_l * PCAPB + cnt, 16)] = jnp.full((16,), d_l, I32)
                    cnt = cnt + 1
                    scnt[pl.ds(b_l * 16, 16)] = jnp.full((16,), cnt, I32)

                    @pl.when(cnt >= FLUSH)
                    def _():
                        fl = sfl[pl.ds(b_l * 16, 16)][0]
                        base = pl.multiple_of((b_l * NW + sid) * CAPW + fl, 8)
                        pltpu.sync_copy(se.at[pl.ds(b_l * PCAPB, FLUSH)],
                                        eix_h.at[pl.ds(base, FLUSH)])
                        pltpu.sync_copy(sd.at[pl.ds(b_l * PCAPB, FLUSH)],
                                        dl_h.at[pl.ds(base, FLUSH)])
                        sfl[pl.ds(b_l * 16, 16)] = jnp.full((16,), fl + FLUSH, I32)
                        scnt[pl.ds(b_l * 16, 16)] = jnp.zeros((16,), I32)
                return carry

            return lax.fori_loop(0, 2000 // 16, vec, carry)

        lax.fori_loop(0, EPW // 2000, chunk, 0)

        def finb(b, carry):
            cnt = scnt[pl.ds(b * 16, 16)][0]
            fl = sfl[pl.ds(b * 16, 16)][0]
            base = pl.multiple_of((b * NW + sid) * CAPW + fl, 8)
            pltpu.sync_copy(se.at[pl.ds(b * PCAPB, FLUSH)],
                            eix_h.at[pl.ds(base, FLUSH)])
            pltpu.sync_copy(sd.at[pl.ds(b * PCAPB, FLUSH)],
                            dl_h.at[pl.ds(base, FLUSH)])
            cbuf[pl.ds(0, 16)] = jnp.full((16,), fl + cnt, I32)
            pltpu.sync_copy(
                cbuf, cnt_h.at[pl.ds(pl.multiple_of((b * NW + sid) * 16, 8), 16)])
            return carry
        lax.fori_loop(0, NW, finb, 0)

    return k(dst)


def _scatter_max(z2, routed):
    eix, dl, cnts = routed
    mesh = plsc.VectorSubcoreMesh(core_axis_name="c", subcore_axis_name="s")
    I32 = jnp.int32

    @functools.partial(
        pl.kernel,
        out_type=jax.ShapeDtypeStruct((NP * D,), jnp.float32),
        mesh=mesh,
        scratch_types=[
            pltpu.VMEM(((NPW + 1) * D,), jnp.float32),
            pltpu.VMEM((512,), I32),
            pltpu.VMEM((128,), I32),
            pltpu.VMEM((128,), I32),
            pltpu.VMEM((128, D), jnp.float32),
            pltpu.SemaphoreType.DMA,
        ],
    )
    def k(z2_h, eix_h, dl_h, cnt_h, out_h, nodebuf, cbuf, ebuf, dbuf, rows,
          sem):
        wid = lax.axis_index("s") * NC + lax.axis_index("c")
        ninf = jnp.full((16,), NEG, jnp.float32)

        def initrow(r, carry):
            nodebuf[pl.ds(16 * r, 16)] = ninf
            return carry
        lax.fori_loop(0, (NPW + 1) * D // 16, initrow, 0)

        pltpu.sync_copy(
            cnt_h.at[pl.ds(pl.multiple_of(wid * NW * 16, 8), 512)], cbuf)

        def seg(s_, carry):
            cnt = cbuf[pl.ds(s_ * 16, 16)][0]
            rbase = (wid * NW + s_) * CAPW
            nb = (cnt + 127) // 128
            lval = (cnt // FLUSH) * FLUSH + FLUSH

            def batch(f, carry):
                start = pl.multiple_of(
                    rbase + jnp.minimum(f * 128, lval - 128), 8)
                pltpu.sync_copy(eix_h.at[pl.ds(start, 128)], ebuf)
                pltpu.sync_copy(dl_h.at[pl.ds(start, 128)], dbuf)
                pltpu.async_copy(z2_h.at[ebuf], rows, sem).wait()

                def grp(t, carry):
                    dlv = dbuf[pl.ds(16 * t, 16)]
                    for l in range(16):
                        off = dlv[l] * D
                        r_ = 16 * t + l
                        for j in range(8):
                            cur = nodebuf[pl.ds(off + 16 * j, 16)]
                            val = rows[r_, pl.ds(16 * j, 16)]
                            nodebuf[pl.ds(off + 16 * j, 16)] = jnp.maximum(cur, val)
                    return carry
                return lax.fori_loop(0, 8, grp, carry)

            return lax.fori_loop(0, nb, batch, carry)

        lax.fori_loop(0, NW, seg, 0)

        pltpu.sync_copy(
            nodebuf.at[pl.ds(0, NPW * D)],
            out_h.at[pl.ds(pl.multiple_of(wid * NPW * D, 8), NPW * D)])

    rows_shape = k(z2, eix, dl, cnts)
    return rows_shape.reshape(NP, D)


def _forward_impl(x, params, edge_index, batch):
    src = edge_index[0]
    dst = edge_index[1]
    xp = jnp.zeros((NP, D), F32).at[:N].set(x)
    oh = jnp.zeros((NP, 128), F32).at[jnp.arange(N), batch].set(1.0)

    routed = _route(dst)

    def conv(table, pre):
        xi, xj = _gather_pair(table, src, dst)
        a1, st1 = _tc_p1(xi, xj, params[pre + '_W1'], params[pre + '_b1'])
        z2, st2 = _tc_p2(a1, st1, params[pre + '_W2'], params[pre + '_b2'],
                         params[pre + '_g1'], params[pre + '_be1'])
        mz = _scatter_max(z2, routed)
        return mz, st2

    m1, st2a = conv(xp, 'c1')
    x1 = _tc_fin(m1, st2a, params['c1_g2'], params['c1_be2'])
    m2, st2b = conv(x1, 'c2')
    seg, st = _tc_tail_main(m2, st2b, params['c2_g2'], params['c2_be2'],
                            x1, oh, params['l1_W'], params['l1_b'])
    return _tc_tail_head(seg, st, params)


@jax.jit
def kernel(x, params, edge_index, batch):
    return _forward_impl(x, params, edge_index, batch)


# speedup vs baseline: 4.3869x; 1.8561x over previous
import functools
import jax
import jax.numpy as jnp
from jax import lax
from jax.experimental import pallas as pl
from jax.experimental.pallas import tpu as pltpu
from jax.experimental.pallas import tpu_sc as plsc

NC = 2
NS = 16
NW = NC * NS

N = 10000
NP = 10016
E = 320000
D = 128
G = 16
EPS = 1e-5
F32 = jnp.float32
BF16 = jnp.bfloat16

EBLK = 512
EGRID = E // EBLK
NBLK = 2504
NGRID = NP // NBLK
M_ = 256


def _dot_bf16(a, b):
    return lax.dot_general(a.astype(BF16), b.astype(BF16),
                           (((1,), (0,)), ((), ())),
                           preferred_element_type=F32)


def _dot_f32(a, b):
    return lax.dot_general(a, b, (((1,), (0,)), ((), ())),
                           precision=lax.Precision.HIGHEST,
                           preferred_element_type=F32)


def _p1_body(xi_ref, xj_ref, w1_ref, b1_ref, a1_ref, st1_ref, acc_ref):
    i = pl.program_id(0)
    xi = xi_ref[...]
    d = xj_ref[...] - xi
    m = jnp.concatenate([xi, d], axis=1)
    a1 = jnp.maximum(_dot_bf16(m, w1_ref[...]) + b1_ref[0, :][None, :], 0.0)
    a1_ref[...] = a1

    ps = jnp.sum(a1.reshape(EBLK // 8, 8, D), axis=0)
    pq = jnp.sum((a1 * a1).reshape(EBLK // 8, 8, D), axis=0)

    @pl.when(i == 0)
    def _():
        acc_ref[...] = jnp.zeros_like(acc_ref)

    acc_ref[0:8, :] += ps
    acc_ref[8:16, :] += pq
    st1_ref[...] = jnp.concatenate([
        jnp.sum(acc_ref[0:8, :], axis=0, keepdims=True),
        jnp.sum(acc_ref[8:16, :], axis=0, keepdims=True),
        jnp.zeros((6, D), F32),
    ], axis=0)


def _tc_p1(xi, xj, w1, b1):
    return pl.pallas_call(
        _p1_body,
        grid=(EGRID,),
        in_specs=[
            pl.BlockSpec((EBLK, D), lambda i: (i, 0)),
            pl.BlockSpec((EBLK, D), lambda i: (i, 0)),
            pl.BlockSpec((2 * D, D), lambda i: (0, 0)),
            pl.BlockSpec((1, D), lambda i: (0, 0)),
        ],
        out_specs=[
            pl.BlockSpec((EBLK, D), lambda i: (i, 0)),
            pl.BlockSpec((8, D), lambda i: (0, 0)),
        ],
        out_shape=[
            jax.ShapeDtypeStruct((E, D), F32),
            jax.ShapeDtypeStruct((8, D), F32),
        ],
        scratch_shapes=[pltpu.VMEM((16, D), F32)],
    )(xi, xj, w1, b1.reshape(1, D))


def _p2_body(a1_ref, st1_ref, w2_ref, b2_ref, g1_ref, be1_ref,
             z2_ref, st2_ref, acc_ref):
    i = pl.program_id(0)

    s_ = st1_ref[0, :] / E
    q_ = st1_ref[1, :] / E
    den = jnp.sqrt(q_ - s_ * s_ + EPS)

    a1 = a1_ref[...]
    h1 = g1_ref[0, :][None, :] * (a1 - s_[None, :]) / den[None, :] \
        + be1_ref[0, :][None, :]
    z2 = _dot_bf16(h1, w2_ref[...]) + b2_ref[0, :][None, :]
    z2_ref[...] = z2

    a2 = jnp.maximum(z2, 0.0)
    ps = jnp.sum(a2.reshape(EBLK // 8, 8, D), axis=0)
    pq = jnp.sum((a2 * a2).reshape(EBLK // 8, 8, D), axis=0)

    @pl.when(i == 0)
    def _():
        acc_ref[...] = jnp.zeros_like(acc_ref)

    acc_ref[0:8, :] += ps
    acc_ref[8:16, :] += pq
    st2_ref[...] = jnp.concatenate([
        jnp.sum(acc_ref[0:8, :], axis=0, keepdims=True),
        jnp.sum(acc_ref[8:16, :], axis=0, keepdims=True),
        jnp.zeros((6, D), F32),
    ], axis=0)


def _tc_p2(a1, stats1, w2, b2, g1, be1):
    return pl.pallas_call(
        _p2_body,
        grid=(EGRID,),
        in_specs=[
            pl.BlockSpec((EBLK, D), lambda i: (i, 0)),
            pl.BlockSpec((8, D), lambda i: (0, 0)),
            pl.BlockSpec((D, D), lambda i: (0, 0)),
            pl.BlockSpec((1, D), lambda i: (0, 0)),
            pl.BlockSpec((1, D), lambda i: (0, 0)),
            pl.BlockSpec((1, D), lambda i: (0, 0)),
        ],
        out_specs=[
            pl.BlockSpec((EBLK, D), lambda i: (i, 0)),
            pl.BlockSpec((8, D), lambda i: (0, 0)),
        ],
        out_shape=[
            jax.ShapeDtypeStruct((E, D), F32),
            jax.ShapeDtypeStruct((8, D), F32),
        ],
        scratch_shapes=[pltpu.VMEM((16, D), F32)],
    )(a1, stats1, w2, b2.reshape(1, D), g1.reshape(1, D), be1.reshape(1, D))


def _fin_body(m_ref, st2_ref, g2_ref, be2_ref, x_ref):
    s_ = st2_ref[0, :] / E
    q_ = st2_ref[1, :] / E
    den = jnp.sqrt(q_ - s_ * s_ + EPS)
    m = m_ref[...]
    bn = g2_ref[0, :][None, :] * (jnp.maximum(m, 0.0) - s_[None, :]) \
        / den[None, :] + be2_ref[0, :][None, :]
    x_ref[...] = jnp.where(m > EMPTY_THRESH, bn, 0.0)


def _tc_fin(mz, stats2, g2, be2):
    return pl.pallas_call(
        _fin_body,
        grid=(NGRID,),
        in_specs=[
            pl.BlockSpec((NBLK, D), lambda i: (i, 0)),
            pl.BlockSpec((8, D), lambda i: (0, 0)),
            pl.BlockSpec((1, D), lambda i: (0, 0)),
            pl.BlockSpec((1, D), lambda i: (0, 0)),
        ],
        out_specs=pl.BlockSpec((NBLK, D), lambda i: (i, 0)),
        out_shape=jax.ShapeDtypeStruct((NP, D), F32),
    )(mz, stats2, g2.reshape(1, D), be2.reshape(1, D))


def _tail_main_body(m_ref, st2_ref, g2_ref, be2_ref, x1_ref, oh_ref,
                    lw_ref, lb_ref, seg_ref, st_ref, segacc_ref, acc_ref):
    i = pl.program_id(0)

    s_ = st2_ref[0, :] / E
    q_ = st2_ref[1, :] / E
    den = jnp.sqrt(q_ - s_ * s_ + EPS)
    m = m_ref[...]
    bn = g2_ref[0, :][None, :] * (jnp.maximum(m, 0.0) - s_[None, :]) \
        / den[None, :] + be2_ref[0, :][None, :]
    x2 = jnp.where(m > EMPTY_THRESH, bn, 0.0)

    cat = jnp.concatenate([x1_ref[...], x2], axis=1)
    opre = jnp.maximum(_dot_bf16(cat, lw_ref[...]) + lb_ref[0, :][None, :], 0.0)

    oh = oh_ref[...]
    rowmask = jnp.sum(oh, axis=1, keepdims=True)
    om = opre * rowmask

    ps = jnp.sum(om.reshape(NBLK // 8, 8, M_), axis=0)
    pq = jnp.sum((om * om).reshape(NBLK // 8, 8, M_), axis=0)
    seg = lax.dot_general(oh, om, (((0,), (0,)), ((), ())),
                          precision=lax.Precision.HIGHEST,
                          preferred_element_type=F32)
    cntp = jnp.sum(oh.reshape(NBLK // 8, 8, 128), axis=0)

    @pl.when(i == 0)
    def _():
        acc_ref[...] = jnp.zeros_like(acc_ref)
        segacc_ref[...] = jnp.zeros_like(segacc_ref)

    acc_ref[0:8, :] += ps
    acc_ref[8:16, :] += pq
    acc_ref[16:24, 0:128] += cntp
    segacc_ref[...] += seg[0:16, :]

    seg_ref[...] = segacc_ref[...]
    st_ref[...] = jnp.concatenate([
        jnp.sum(acc_ref[0:8, :], axis=0, keepdims=True),
        jnp.sum(acc_ref[8:16, :], axis=0, keepdims=True),
        jnp.sum(acc_ref[16:24, :], axis=0, keepdims=True),
        jnp.zeros((5, M_), F32),
    ], axis=0)


def _tc_tail_main(mz2, stats2, g2, be2, x1, oh, lw, lb):
    return pl.pallas_call(
        _tail_main_body,
        grid=(NGRID,),
        in_specs=[
            pl.BlockSpec((NBLK, D), lambda i: (i, 0)),
            pl.BlockSpec((8, D), lambda i: (0, 0)),
            pl.BlockSpec((1, D), lambda i: (0, 0)),
            pl.BlockSpec((1, D), lambda i: (0, 0)),
            pl.BlockSpec((NBLK, D), lambda i: (i, 0)),
            pl.BlockSpec((NBLK, 128), lambda i: (i, 0)),
            pl.BlockSpec((2 * D, M_), lambda i: (0, 0)),
            pl.BlockSpec((1, M_), lambda i: (0, 0)),
        ],
        out_specs=[
            pl.BlockSpec((16, M_), lambda i: (0, 0)),
            pl.BlockSpec((8, M_), lambda i: (0, 0)),
        ],
        out_shape=[
            jax.ShapeDtypeStruct((16, M_), F32),
            jax.ShapeDtypeStruct((8, M_), F32),
        ],
        scratch_shapes=[pltpu.VMEM((16, M_), F32), pltpu.VMEM((24, M_), F32)],
    )(mz2, stats2, g2.reshape(1, D), be2.reshape(1, D), x1, oh, lw,
      lb.reshape(1, M_))


def _tail_head_body(seg_ref, st_ref, lg_ref, lbe_ref,
                    m1w_ref, m1b_ref, m1g_ref, m1be_ref,
                    m2w_ref, m2b_ref, m2g_ref, m2be_ref,
                    mfw_ref, mfb_ref, y_ref):
    s_ = st_ref[0, :] / N
    q_ = st_ref[1, :] / N
    den = jnp.sqrt(q_ - s_ * s_ + EPS)
    cnt = st_ref[2, 0:16].reshape(16, 1)
    seg = seg_ref[...]
    mean_o = seg / jnp.maximum(cnt, 1.0)
    bnp = lg_ref[0, :][None, :] * (mean_o - s_[None, :]) / den[None, :] \
        + lbe_ref[0, :][None, :]
    pooled = jnp.where(cnt > 0.0, bnp, 0.0)

    def blk(xx, w, b, g, be):
        a = jnp.maximum(_dot_bf16(xx, w) + b[0, :][None, :], 0.0)
        mm = jnp.mean(a, axis=0, keepdims=True)
        vv = jnp.mean((a - mm) ** 2, axis=0, keepdims=True)
        return g[0, :][None, :] * (a - mm) / jnp.sqrt(vv + EPS) + be[0, :][None, :]

    h = blk(pooled, m1w_ref[...], m1b_ref, m1g_ref, m1be_ref)
    h = blk(h, m2w_ref[...], m2b_ref, m2g_ref, m2be_ref)
    hb = h.astype(BF16).astype(F32)
    wb = mfw_ref[...][:, 0].astype(BF16).astype(F32)
    y = jnp.sum(hb * wb[None, :], axis=1, keepdims=True) + mfb_ref[0, 0]
    y_ref[...] = jnp.broadcast_to(y, (16, 128))


def _tc_tail_head(seg, st, p):
    r1 = lambda a: a.reshape(1, -1)
    out = pl.pallas_call(
        _tail_head_body,
        out_shape=jax.ShapeDtypeStruct((16, 128), F32),
    )(seg, st, r1(p['l1_g']), r1(p['l1_be']),
      p['m1_W'], r1(p['m1_b']), r1(p['m1_g']), r1(p['m1_be']),
      p['m2_W'], r1(p['m2_b']), r1(p['m2_g']), r1(p['m2_be']),
      p['mf_W'], r1(p['mf_b']))
    return out[:, 0]


CB = 128
CHUNKS = E // CB
WPASS = (CHUNKS + NW - 1) // NW


def _gather_pair(table, src, dst):
    mesh = plsc.VectorSubcoreMesh(core_axis_name="c", subcore_axis_name="s")

    @functools.partial(
        pl.kernel,
        out_type=[jax.ShapeDtypeStruct((E, D), F32),
                  jax.ShapeDtypeStruct((E, D), F32)],
        mesh=mesh,
        scratch_types=[
            pltpu.VMEM((CB,), jnp.int32), pltpu.VMEM((CB,), jnp.int32),
            pltpu.VMEM((CB, D), F32), pltpu.VMEM((CB, D), F32),
            pltpu.SemaphoreType.DMA, pltpu.SemaphoreType.DMA,
        ],
    )
    def k(table_h, dst_h, src_h, xi_h, xj_h, idx_d, idx_s, bi, bj, sem1, sem2):
        wid = lax.axis_index("s") * NC + lax.axis_index("c")

        def body(t, carry):
            cid = wid + NW * t

            @pl.when(cid < CHUNKS)
            def _():
                base = cid * CB
                pltpu.sync_copy(dst_h.at[pl.ds(base, CB)], idx_d)
                pltpu.sync_copy(src_h.at[pl.ds(base, CB)], idx_s)
                c1 = pltpu.async_copy(table_h.at[idx_d], bi, sem1)
                c2 = pltpu.async_copy(table_h.at[idx_s], bj, sem2)
                c1.wait()
                c2.wait()
                pltpu.sync_copy(bi, xi_h.at[pl.ds(base, CB)])
                pltpu.sync_copy(bj, xj_h.at[pl.ds(base, CB)])

            return carry

        lax.fori_loop(0, WPASS, body, 0)

    return k(table, dst, src)


NPW = NP // NW
EPW = E // NW
CAPW = 10112
PCAPB = 144
FLUSH = 112
NEG = -3.0e38
EMPTY_THRESH = -1.0e38


def _route(dst):
    mesh = plsc.VectorSubcoreMesh(core_axis_name="c", subcore_axis_name="s")
    iota16 = lambda: lax.broadcasted_iota(jnp.int32, (16,), 0)
    I32 = jnp.int32

    @functools.partial(
        pl.kernel,
        out_type=[
            jax.ShapeDtypeStruct((NW * NW * CAPW,), I32),
            jax.ShapeDtypeStruct((NW * NW * CAPW,), I32),
            jax.ShapeDtypeStruct((NW * NW * 16,), I32),
        ],
        mesh=mesh,
        scratch_types=[
            pltpu.VMEM((2000,), I32),
            pltpu.VMEM((NW * PCAPB,), I32),
            pltpu.VMEM((NW * PCAPB,), I32),
            pltpu.VMEM((NW * 16,), I32),
            pltpu.VMEM((NW * 16,), I32),
            pltpu.VMEM((16,), I32),
        ],
    )
    def k(dst_h, eix_h, dl_h, cnt_h, dbuf, se, sd, scnt, sfl, cbuf):
        sid = lax.axis_index("s") * NC + lax.axis_index("c")
        zero16 = jnp.zeros((16,), I32)
        trash16 = jnp.full((16,), NPW, I32)

        def initb(b, carry):
            for j in range(PCAPB // 16):
                se[pl.ds(b * PCAPB + 16 * j, 16)] = zero16
                sd[pl.ds(b * PCAPB + 16 * j, 16)] = trash16
            scnt[pl.ds(b * 16, 16)] = zero16
            sfl[pl.ds(b * 16, 16)] = zero16
            return carry
        lax.fori_loop(0, NW, initb, 0)

        def chunk(c, carry):
            cbase = pl.multiple_of(sid * EPW + c * 2000, 8)
            pltpu.sync_copy(dst_h.at[pl.ds(cbase, 2000)], dbuf)

            def vec(kk, carry):
                dv = dbuf[pl.ds(16 * kk, 16)]
                fv = dv.astype(jnp.float32) + 0.5
                bv = (fv * (1.0 / NPW)).astype(I32)
                dlv = dv - bv * NPW
                gbase = cbase + 16 * kk

                for l in range(16):
                    b_l = bv[l]
                    d_l = dlv[l]
                    cnt = scnt[pl.ds(b_l * 16, 16)][0]
                    se[pl.ds(b_l * PCAPB + cnt, 16)] = jnp.full((16,), gbase + l, I32)
                    sd[pl.ds(b_l * PCAPB + cnt, 16)] = jnp.full((16,), d_l, I32)
                    cnt = cnt + 1
                    scnt[pl.ds(b_l * 16, 16)] = jnp.full((16,), cnt, I32)

                    @pl.when(cnt >= FLUSH)
                    def _():
                        fl = sfl[pl.ds(b_l * 16, 16)][0]
                        base = pl.multiple_of((b_l * NW + sid) * CAPW + fl, 8)
                        pltpu.sync_copy(se.at[pl.ds(b_l * PCAPB, FLUSH)],
                                        eix_h.at[pl.ds(base, FLUSH)])
                        pltpu.sync_copy(sd.at[pl.ds(b_l * PCAPB, FLUSH)],
                                        dl_h.at[pl.ds(base, FLUSH)])
                        sfl[pl.ds(b_l * 16, 16)] = jnp.full((16,), fl + FLUSH, I32)
                        scnt[pl.ds(b_l * 16, 16)] = jnp.zeros((16,), I32)
                        for j in range(2):
                            ve = se[pl.ds(b_l * PCAPB + FLUSH + 16 * j, 16)]
                            vd = sd[pl.ds(b_l * PCAPB + FLUSH + 16 * j, 16)]
                            se[pl.ds(b_l * PCAPB + 16 * j, 16)] = ve
                            sd[pl.ds(b_l * PCAPB + 16 * j, 16)] = vd
                return carry

            return lax.fori_loop(0, 2000 // 16, vec, carry)

        lax.fori_loop(0, EPW // 2000, chunk, 0)

        def finb(b, carry):
            cnt = scnt[pl.ds(b * 16, 16)][0]
            fl = sfl[pl.ds(b * 16, 16)][0]
            base = pl.multiple_of((b * NW + sid) * CAPW + fl, 8)
            pltpu.sync_copy(se.at[pl.ds(b * PCAPB, 128)],
                            eix_h.at[pl.ds(base, 128)])
            pltpu.sync_copy(sd.at[pl.ds(b * PCAPB, 128)],
                            dl_h.at[pl.ds(base, 128)])
            cbuf[pl.ds(0, 16)] = jnp.full((16,), fl + cnt, I32)
            pltpu.sync_copy(
                cbuf, cnt_h.at[pl.ds(pl.multiple_of((b * NW + sid) * 16, 8), 16)])
            return carry
        lax.fori_loop(0, NW, finb, 0)

    return k(dst)


def _scatter_max(z2, routed):
    eix, dl, cnts = routed
    mesh = plsc.VectorSubcoreMesh(core_axis_name="c", subcore_axis_name="s")
    I32 = jnp.int32

    @functools.partial(
        pl.kernel,
        out_type=jax.ShapeDtypeStruct((NP * D,), jnp.float32),
        mesh=mesh,
        scratch_types=[
            pltpu.VMEM(((NPW + 1) * D,), jnp.float32),
            pltpu.VMEM((512,), I32),
            pltpu.VMEM((128,), I32),
            pltpu.VMEM((128,), I32),
            pltpu.VMEM((128, D), jnp.float32),
            pltpu.SemaphoreType.DMA,
        ],
    )
    def k(z2_h, eix_h, dl_h, cnt_h, out_h, nodebuf, cbuf, ebuf, dbuf, rows,
          sem):
        wid = lax.axis_index("s") * NC + lax.axis_index("c")
        ninf = jnp.full((16,), NEG, jnp.float32)

        def initrow(r, carry):
            nodebuf[pl.ds(16 * r, 16)] = ninf
            return carry
        lax.fori_loop(0, (NPW + 1) * D // 16, initrow, 0)

        pltpu.sync_copy(
            cnt_h.at[pl.ds(pl.multiple_of(wid * NW * 16, 8), 512)], cbuf)

        def seg(s_, carry):
            cnt = cbuf[pl.ds(s_ * 16, 16)][0]
            rbase = (wid * NW + s_) * CAPW
            nb = (cnt + 127) // 128
            lval = (cnt // FLUSH) * FLUSH + 128

            def batch(f, carry):
                start = pl.multiple_of(
                    rbase + jnp.minimum(f * 128, lval - 128), 8)
                pltpu.sync_copy(eix_h.at[pl.ds(start, 128)], ebuf)
                pltpu.sync_copy(dl_h.at[pl.ds(start, 128)], dbuf)
                pltpu.async_copy(z2_h.at[ebuf], rows, sem).wait()

                def grp(t, carry):
                    dlv = dbuf[pl.ds(16 * t, 16)]
                    for l in range(16):
                        off = dlv[l] * D
                        r_ = 16 * t + l
                        for j in range(8):
                            cur = nodebuf[pl.ds(off + 16 * j, 16)]
                            val = rows[r_, pl.ds(16 * j, 16)]
                            nodebuf[pl.ds(off + 16 * j, 16)] = jnp.maximum(cur, val)
                    return carry
                return lax.fori_loop(0, 8, grp, carry)

            return lax.fori_loop(0, nb, batch, carry)

        lax.fori_loop(0, NW, seg, 0)

        pltpu.sync_copy(
            nodebuf.at[pl.ds(0, NPW * D)],
            out_h.at[pl.ds(pl.multiple_of(wid * NPW * D, 8), NPW * D)])

    rows_shape = k(z2, eix, dl, cnts)
    return rows_shape.reshape(NP, D)


def _forward_impl(x, params, edge_index, batch):
    src = edge_index[0]
    dst = edge_index[1]
    xp = jnp.zeros((NP, D), F32).at[:N].set(x)
    oh = jnp.zeros((NP, 128), F32).at[jnp.arange(N), batch].set(1.0)

    routed = _route(dst)

    def conv(table, pre):
        xi, xj = _gather_pair(table, src, dst)
        a1, st1 = _tc_p1(xi, xj, params[pre + '_W1'], params[pre + '_b1'])
        z2, st2 = _tc_p2(a1, st1, params[pre + '_W2'], params[pre + '_b2'],
                         params[pre + '_g1'], params[pre + '_be1'])
        mz = _scatter_max(z2, routed)
        return mz, st2

    m1, st2a = conv(xp, 'c1')
    x1 = _tc_fin(m1, st2a, params['c1_g2'], params['c1_be2'])
    m2, st2b = conv(x1, 'c2')
    seg, st = _tc_tail_main(m2, st2b, params['c2_g2'], params['c2_be2'],
                            x1, oh, params['l1_W'], params['l1_b'])
    return _tc_tail_head(seg, st, params)


@jax.jit
def kernel(x, params, edge_index, batch):
    return _forward_impl(x, params, edge_index, batch)
